# baseline (speedup 1.0000x reference)
import functools

import jax
import jax.numpy as jnp
from jax import lax
from jax.experimental import pallas as pl
from jax.experimental.pallas import tpu as pltpu
from jax.experimental.pallas import tpu_sc as plsc

BATCH = 4096
SEQ = 200
DIM = 64
TOTAL = BATCH * SEQ
NUM_WORKERS = 32
PER_WORKER = TOTAL // NUM_WORKERS
IDX_MINOR = 128
CHUNK = 512
SUB = CHUNK // IDX_MINOR
NCHUNK = PER_WORKER // CHUNK

_mesh = plsc.VectorSubcoreMesh(core_axis_name="c", subcore_axis_name="s")


@functools.partial(
    pl.kernel,
    mesh=_mesh,
    compiler_params=pltpu.CompilerParams(use_tc_tiling_on_sc=False),
    out_type=jax.ShapeDtypeStruct((TOTAL, DIM), jnp.float32),
    scratch_types=[
        pltpu.VMEM((PER_WORKER // IDX_MINOR, IDX_MINOR), jnp.int32),
        pltpu.VMEM((CHUNK, DIM), jnp.float32),
        pltpu.SemaphoreType.DMA,
    ],
)
def _gather_kernel(idx_hbm, table_hbm, out_hbm, idx_v, rows_v, sem):
    wid = lax.axis_index("s") * 2 + lax.axis_index("c")
    row_base = wid * (PER_WORKER // IDX_MINOR)
    pltpu.sync_copy(idx_hbm.at[pl.ds(row_base, PER_WORKER // IDX_MINOR)], idx_v)

    def body(j, carry):
        for k in range(SUB):
            cp = pltpu.make_async_copy(
                table_hbm.at[idx_v.at[j * SUB + k]],
                rows_v.at[pl.ds(k * IDX_MINOR, IDX_MINOR), :],
                sem,
            )
            cp.start()
        for k in range(SUB):
            pltpu.make_async_copy(
                table_hbm.at[idx_v.at[j * SUB + k]],
                rows_v.at[pl.ds(k * IDX_MINOR, IDX_MINOR), :],
                sem,
            ).wait()
        out_base = wid * PER_WORKER + j * CHUNK
        pltpu.sync_copy(rows_v, out_hbm.at[pl.ds(out_base, CHUNK), :])
        return carry

    lax.fori_loop(0, NCHUNK, body, 0)


def kernel(input, table):
    idx = input.reshape(TOTAL // IDX_MINOR, IDX_MINOR).astype(jnp.int32)
    out = _gather_kernel(idx, table)
    return out.reshape(BATCH, SEQ, DIM)


# speedup vs baseline: 1.0244x; 1.0244x over previous
import functools

import jax
import jax.numpy as jnp
from jax import lax
from jax.experimental import pallas as pl
from jax.experimental.pallas import tpu as pltpu
from jax.experimental.pallas import tpu_sc as plsc

BATCH = 4096
SEQ = 200
DIM = 64
TOTAL = BATCH * SEQ
NUM_WORKERS = 32
PER_WORKER = TOTAL // NUM_WORKERS
IDX_MINOR = 128
CHUNK = 512
SUB = CHUNK // IDX_MINOR
NCHUNK = PER_WORKER // CHUNK

_mesh = plsc.VectorSubcoreMesh(core_axis_name="c", subcore_axis_name="s")


@functools.partial(
    pl.kernel,
    mesh=_mesh,
    compiler_params=pltpu.CompilerParams(use_tc_tiling_on_sc=False),
    out_type=jax.ShapeDtypeStruct((TOTAL, DIM), jnp.float32),
    scratch_types=[
        pltpu.VMEM((PER_WORKER // IDX_MINOR, IDX_MINOR), jnp.int32),
        pltpu.VMEM((2, CHUNK, DIM), jnp.float32),
        pltpu.SemaphoreType.DMA,
        pltpu.SemaphoreType.DMA,
        pltpu.SemaphoreType.DMA,
    ],
)
def _gather_kernel(idx_hbm, table_hbm, out_hbm, idx_v, rows_v, gsem, ssem0, ssem1):
    ssems = [ssem0, ssem1]
    wid = lax.axis_index("s") * 2 + lax.axis_index("c")
    row_base = wid * (PER_WORKER // IDX_MINOR)
    out_base = wid * PER_WORKER
    pltpu.sync_copy(idx_hbm.at[pl.ds(row_base, PER_WORKER // IDX_MINOR)], idx_v)

    def fire(j, b):
        for k in range(SUB):
            pltpu.make_async_copy(
                table_hbm.at[idx_v.at[j * SUB + k]],
                rows_v.at[b].at[pl.ds(k * IDX_MINOR, IDX_MINOR), :],
                gsem,
            ).start()

    def drain(j, b):
        for k in range(SUB):
            pltpu.make_async_copy(
                table_hbm.at[idx_v.at[j * SUB + k]],
                rows_v.at[b].at[pl.ds(k * IDX_MINOR, IDX_MINOR), :],
                gsem,
            ).wait()

    def store_cp(j, b):
        return pltpu.make_async_copy(
            rows_v.at[b], out_hbm.at[pl.ds(out_base + j * CHUNK, CHUNK), :],
            ssems[b],
        )

    fire(0, 0)
    fire(1, 1)
    drain(0, 0)
    store_cp(0, 0).start()
    drain(1, 1)
    store_cp(1, 1).start()

    def body(i, carry):
        j0 = 2 * i
        j1 = j0 + 1
        store_cp(j0, 0).wait()
        fire(j0, 0)
        store_cp(j1, 1).wait()
        fire(j1, 1)
        drain(j0, 0)
        store_cp(j0, 0).start()
        drain(j1, 1)
        store_cp(j1, 1).start()
        return carry

    lax.fori_loop(1, NCHUNK // 2, body, 0)
    store_cp(NCHUNK - 2, 0).wait()
    store_cp(NCHUNK - 1, 1).wait()


def kernel(input, table):
    idx = input.reshape(TOTAL // IDX_MINOR, IDX_MINOR).astype(jnp.int32)
    out = _gather_kernel(idx, table)
    return out.reshape(BATCH, SEQ, DIM)
